# Initial kernel scaffold; baseline (speedup 1.0000x reference)
import jax
import jax.numpy as jnp
from jax.experimental import pallas as pl


def kernel(x, edge_index, edge_weights, W1, b1, W2, b2):
    raise NotImplementedError("write your pallas kernel here")


# speedup vs baseline: 8.0536x; 8.0536x over previous
import functools

import jax
import jax.numpy as jnp
from jax import lax
from jax.experimental import pallas as pl
from jax.experimental.pallas import tpu as pltpu
from jax.experimental.pallas import tpu_sc as plsc

N = 10000
D = 128
E = 320000

L = 16
NC = 2
NS = 16
CH = 128
N_PAD = 10240
RPT = N_PAD // NS
E_PAD = 323584
FPV = D // L

_mesh = plsc.VectorSubcoreMesh(
    core_axis_name="c", subcore_axis_name="s", num_cores=NC, num_subcores=NS)


def _norm_body(src_ref, dst_ref, ew_ref, norm_ref, c_ref,
               deg_sp, dinv_sp, deg_v, tmp_v, acc_v, dinv_v, c_v, dinvf_v,
               srcc, dstc, ewc, normc):
    cid = lax.axis_index("c")
    sid = lax.axis_index("s")
    zero16 = jnp.zeros((L,), jnp.float32)

    def _zero_deg(i, _):
        deg_v[pl.ds(pl.multiple_of(i * L, L), L)] = zero16
        return 0
    lax.fori_loop(0, N_PAD // L, _zero_deg, 0)

    ept1 = E_PAD // NS
    def _deg_chunk(i, _):
        base = sid * ept1 + i * CH
        pltpu.sync_copy(dst_ref.at[pl.ds(base, CH)], dstc)
        pltpu.sync_copy(ew_ref.at[pl.ds(base, CH)], ewc)
        for j in range(CH // L):
            idx = dstc[pl.ds(j * L, L)]
            w = ewc[pl.ds(j * L, L)]
            plsc.addupdate_scatter(deg_v, [idx], w)
        return 0
    lax.fori_loop(0, ept1 // CH, _deg_chunk, 0)

    pltpu.sync_copy(deg_v, deg_sp.at[sid])
    plsc.subcore_barrier()

    rbase = sid * RPT
    def _zero_acc(i, _):
        acc_v[pl.ds(pl.multiple_of(i * L, L), L)] = zero16
        return 0
    lax.fori_loop(0, RPT // L, _zero_acc, 0)

    def _merge(t, _):
        pltpu.sync_copy(deg_sp.at[t, pl.ds(rbase, RPT)], tmp_v)
        def _add(i, _2):
            sl = pl.ds(pl.multiple_of(i * L, L), L)
            acc_v[sl] = acc_v[sl] + tmp_v[sl]
            return 0
        lax.fori_loop(0, RPT // L, _add, 0)
        return 0
    lax.fori_loop(0, NS, _merge, 0)

    def _newton(i, _):
        sl = pl.ds(pl.multiple_of(i * L, L), L)
        d = acc_v[sl] + 1.0
        bits = plsc.bitcast(d, jnp.int32)
        y = plsc.bitcast(jnp.int32(0x5F3759DF) - (bits >> 1), jnp.float32)
        for _it in range(3):
            y = y * (1.5 - 0.5 * d * y * y)
        dinv_v[sl] = y
        c_v[sl] = y * y
        return 0
    lax.fori_loop(0, RPT // L, _newton, 0)

    pltpu.sync_copy(dinv_v, dinv_sp.at[pl.ds(rbase, RPT)])

    @pl.when(cid == 0)
    def _():
        pltpu.sync_copy(c_v, c_ref.at[pl.ds(rbase, RPT)])

    plsc.subcore_barrier()

    pltpu.sync_copy(dinv_sp, dinvf_v)
    wid = cid * NS + sid
    ept3 = E_PAD // (NC * NS)
    def _norm_chunk(i, _):
        base = wid * ept3 + i * CH
        pltpu.sync_copy(src_ref.at[pl.ds(base, CH)], srcc)
        pltpu.sync_copy(dst_ref.at[pl.ds(base, CH)], dstc)
        pltpu.sync_copy(ew_ref.at[pl.ds(base, CH)], ewc)
        for j in range(CH // L):
            sl = pl.ds(j * L, L)
            a = plsc.load_gather(dinvf_v, [srcc[sl]])
            b = plsc.load_gather(dinvf_v, [dstc[sl]])
            normc[sl] = a * ewc[sl] * b
        pltpu.sync_copy(normc, norm_ref.at[pl.ds(base, CH)])
        return 0
    lax.fori_loop(0, ept3 // CH, _norm_chunk, 0)


_norm_kernel = pl.kernel(
    _norm_body,
    out_type=(jax.ShapeDtypeStruct((E_PAD,), jnp.float32),
              jax.ShapeDtypeStruct((N_PAD,), jnp.float32)),
    mesh=_mesh,
    compiler_params=pltpu.CompilerParams(needs_layout_passes=False),
    scratch_types=[
        pltpu.VMEM_SHARED((NS, N_PAD), jnp.float32),
        pltpu.VMEM_SHARED((N_PAD,), jnp.float32),
        pltpu.VMEM((N_PAD,), jnp.float32),
        pltpu.VMEM((RPT,), jnp.float32),
        pltpu.VMEM((RPT,), jnp.float32),
        pltpu.VMEM((RPT,), jnp.float32),
        pltpu.VMEM((RPT,), jnp.float32),
        pltpu.VMEM((N_PAD,), jnp.float32),
        pltpu.VMEM((CH,), jnp.int32),
        pltpu.VMEM((CH,), jnp.int32),
        pltpu.VMEM((CH,), jnp.float32),
        pltpu.VMEM((CH,), jnp.float32),
    ],
)


def _agg_body(xw_ref, src_ref, dst_ref, norm_ref, out_ref,
              agg_sp, rows_v, srcc, dstc, normc, sem):
    cid = lax.axis_index("c")
    sid = lax.axis_index("s")
    zero16 = jnp.zeros((L,), jnp.float32)

    def _zero_rows(r, _):
        for f in range(FPV):
            rows_v[r, pl.ds(f * L, L)] = zero16
        return 0
    lax.fori_loop(0, CH, _zero_rows, 0)
    for k in range(RPT // CH):
        pltpu.sync_copy(rows_v, agg_sp.at[pl.ds((sid * (RPT // CH) + k) * CH, CH)])
    plsc.subcore_barrier()

    half = E_PAD // NC
    ept = half // NS
    def _edge_chunk(i, _):
        base = cid * half + sid * ept + i * CH
        pltpu.sync_copy(src_ref.at[pl.ds(base, CH)], srcc)
        pltpu.sync_copy(dst_ref.at[pl.ds(base, CH)], dstc)
        pltpu.sync_copy(norm_ref.at[pl.ds(base, CH)], normc)
        pltpu.async_copy(xw_ref.at[srcc], rows_v, sem).wait()
        def _scale(j, _2):
            nvv = normc[pl.ds(pl.multiple_of(j * L, L), L)]
            for rr in range(L):
                nv = nvv[rr]
                r = j * L + rr
                for f in range(FPV):
                    sl = pl.ds(f * L, L)
                    rows_v[r, sl] = rows_v[r, sl] * nv
            return 0
        lax.fori_loop(0, CH // L, _scale, 0)
        pltpu.sync_copy(rows_v, agg_sp.at[dstc], add=True)
        return 0
    lax.fori_loop(0, ept // CH, _edge_chunk, 0)
    plsc.subcore_barrier()

    pltpu.sync_copy(agg_sp.at[pl.ds(sid * RPT, RPT)],
                    out_ref.at[cid, pl.ds(sid * RPT, RPT)])


_agg_kernel = pl.kernel(
    _agg_body,
    out_type=jax.ShapeDtypeStruct((NC, N_PAD, D), jnp.float32),
    mesh=_mesh,
    compiler_params=pltpu.CompilerParams(needs_layout_passes=False),
    scratch_types=[
        pltpu.VMEM_SHARED((N_PAD, D), jnp.float32),
        pltpu.VMEM((CH, D), jnp.float32),
        pltpu.VMEM((CH,), jnp.int32),
        pltpu.VMEM((CH,), jnp.int32),
        pltpu.VMEM((CH,), jnp.float32),
        pltpu.SemaphoreType.DMA,
    ],
)


_R = 400


def _mm_body(x_ref, w_ref, o_ref):
    o_ref[...] = jnp.dot(x_ref[...], w_ref[...],
                         preferred_element_type=jnp.float32)


def _tc_matmul(x, W):
    return pl.pallas_call(
        _mm_body,
        grid=(N // _R,),
        in_specs=[pl.BlockSpec((_R, D), lambda i: (i, 0)),
                  pl.BlockSpec((D, D), lambda i: (0, 0))],
        out_specs=pl.BlockSpec((_R, D), lambda i: (i, 0)),
        out_shape=jax.ShapeDtypeStruct((N, D), jnp.float32),
    )(x, W)


def _combine_mm_body(agg_ref, xw_ref, c_ref, b_ref, w_ref, o_ref):
    h = agg_ref[0] + agg_ref[1] + c_ref[...] * xw_ref[...] + b_ref[...]
    h = jnp.maximum(h, 0.0)
    o_ref[...] = jnp.dot(h, w_ref[...], preferred_element_type=jnp.float32)


def _combine_mm(agg, xw, c, b, W):
    return pl.pallas_call(
        _combine_mm_body,
        grid=(N // _R,),
        in_specs=[pl.BlockSpec((NC, _R, D), lambda i: (0, i, 0)),
                  pl.BlockSpec((_R, D), lambda i: (i, 0)),
                  pl.BlockSpec((_R, 1), lambda i: (i, 0)),
                  pl.BlockSpec((1, D), lambda i: (0, 0)),
                  pl.BlockSpec((D, D), lambda i: (0, 0))],
        out_specs=pl.BlockSpec((_R, D), lambda i: (i, 0)),
        out_shape=jax.ShapeDtypeStruct((N, D), jnp.float32),
    )(agg, xw, c, b, W)


def _final_body(agg_ref, xw_ref, c_ref, b_ref, o_ref):
    o_ref[...] = agg_ref[0] + agg_ref[1] + c_ref[...] * xw_ref[...] + b_ref[...]


def _final(agg, xw, c, b):
    return pl.pallas_call(
        _final_body,
        grid=(N // _R,),
        in_specs=[pl.BlockSpec((NC, _R, D), lambda i: (0, i, 0)),
                  pl.BlockSpec((_R, D), lambda i: (i, 0)),
                  pl.BlockSpec((_R, 1), lambda i: (i, 0)),
                  pl.BlockSpec((1, D), lambda i: (0, 0))],
        out_specs=pl.BlockSpec((_R, D), lambda i: (i, 0)),
        out_shape=jax.ShapeDtypeStruct((N, D), jnp.float32),
    )(agg, xw, c, b)


def kernel(x, edge_index, edge_weights, W1, b1, W2, b2):
    pad = E_PAD - E
    src_p = jnp.concatenate([edge_index[0], jnp.zeros((pad,), jnp.int32)])
    dst_p = jnp.concatenate([edge_index[1], jnp.zeros((pad,), jnp.int32)])
    ew_p = jnp.concatenate([edge_weights, jnp.zeros((pad,), jnp.float32)])

    norm, c = _norm_kernel(src_p, dst_p, ew_p)
    c2 = c[:N, None]
    b1r = b1[None, :]
    b2r = b2[None, :]

    xw1 = _tc_matmul(x, W1)
    agg1 = _agg_kernel(xw1, src_p, dst_p, norm)
    xw2 = _combine_mm(agg1, xw1, c2, b1r, W2)
    agg2 = _agg_kernel(xw2, src_p, dst_p, norm)
    return _final(agg2, xw2, c2, b2r)


# speedup vs baseline: 10.2919x; 1.2779x over previous
import functools

import jax
import jax.numpy as jnp
from jax import lax
from jax.experimental import pallas as pl
from jax.experimental.pallas import tpu as pltpu
from jax.experimental.pallas import tpu_sc as plsc

N = 10000
D = 128
E = 320000

L = 16
NC = 2
NS = 16
CH = 128
N_PAD = 10240
RPT = N_PAD // NS
E_PAD = 327680
NCHT = E_PAD // CH
NCH = NCHT // (NC * NS)
NB = 4
NG = NCH // NB
FPV = D // L

_mesh = plsc.VectorSubcoreMesh(
    core_axis_name="c", subcore_axis_name="s", num_cores=NC, num_subcores=NS)
_sc_params = pltpu.CompilerParams(needs_layout_passes=False)


def _dinv_body(dst_ref, ew_ref, dinv_ref,
               deg_sp, deg_v, tmp_v, acc_v, dinv_v, dst1_v, ew1_v):
    cid = lax.axis_index("c")
    sid = lax.axis_index("s")
    zero16 = jnp.zeros((L,), jnp.float32)

    def _zero_deg(i, _):
        deg_v[pl.ds(pl.multiple_of(i * L, L), L)] = zero16
        return 0
    lax.fori_loop(0, N_PAD // L, _zero_deg, 0)

    ept1 = E_PAD // NS
    pltpu.sync_copy(dst_ref.at[pl.ds(sid * ept1, ept1)], dst1_v)
    pltpu.sync_copy(ew_ref.at[pl.ds(sid * ept1, ept1)], ew1_v)

    def _deg_step(i, _):
        sl = pl.ds(pl.multiple_of(i * L, L), L)
        plsc.addupdate_scatter(deg_v, [dst1_v[sl]], ew1_v[sl])
        return 0
    lax.fori_loop(0, ept1 // L, _deg_step, 0)

    pltpu.sync_copy(deg_v, deg_sp.at[sid])
    plsc.subcore_barrier()

    rbase = sid * RPT
    def _zero_acc(i, _):
        acc_v[pl.ds(pl.multiple_of(i * L, L), L)] = zero16
        return 0
    lax.fori_loop(0, RPT // L, _zero_acc, 0)

    def _merge(t, _):
        pltpu.sync_copy(deg_sp.at[t, pl.ds(rbase, RPT)], tmp_v)
        def _add(i, _2):
            sl = pl.ds(pl.multiple_of(i * L, L), L)
            acc_v[sl] = acc_v[sl] + tmp_v[sl]
            return 0
        lax.fori_loop(0, RPT // L, _add, 0)
        return 0
    lax.fori_loop(0, NS, _merge, 0)

    def _newton(i, _):
        sl = pl.ds(pl.multiple_of(i * L, L), L)
        d = acc_v[sl] + 1.0
        bits = plsc.bitcast(d, jnp.int32)
        y = plsc.bitcast(jnp.int32(0x5F3759DF) - (bits >> 1), jnp.float32)
        for _it in range(3):
            y = y * (1.5 - 0.5 * d * y * y)
        dinv_v[sl] = y
        return 0
    lax.fori_loop(0, RPT // L, _newton, 0)

    @pl.when(cid == 0)
    def _():
        pltpu.sync_copy(dinv_v, dinv_ref.at[pl.ds(rbase, RPT)])


_dinv_kernel = pl.kernel(
    _dinv_body,
    out_type=jax.ShapeDtypeStruct((N_PAD,), jnp.float32),
    mesh=_mesh,
    compiler_params=_sc_params,
    scratch_types=[
        pltpu.VMEM_SHARED((NS, N_PAD), jnp.float32),
        pltpu.VMEM((N_PAD,), jnp.float32),
        pltpu.VMEM((RPT,), jnp.float32),
        pltpu.VMEM((RPT,), jnp.float32),
        pltpu.VMEM((RPT,), jnp.float32),
        pltpu.VMEM((E_PAD // NS,), jnp.int32),
        pltpu.VMEM((E_PAD // NS,), jnp.float32),
    ],
)


def _agg_body(xw_ref, edges_ref, out_ref,
              agg_sp, r0, r1, e0, e1, e2, e3,
              g0, g1, s0, s1, es0, es1, es2, es3):
    cid = lax.axis_index("c")
    sid = lax.axis_index("s")
    rows = [r0, r1]
    ebuf = [e0, e1, e2, e3]
    gsem = [g0, g1]
    ssem = [s0, s1]
    esem = [es0, es1, es2, es3]
    zero16 = jnp.zeros((L,), jnp.float32)

    def _zero_rows(r, _):
        for f in range(FPV):
            r0[r, pl.ds(f * L, L)] = zero16
        return 0
    lax.fori_loop(0, CH, _zero_rows, 0)
    for k in range(RPT // CH):
        pltpu.sync_copy(r0, agg_sp.at[pl.ds((sid * (RPT // CH) + k) * CH, CH)])
    plsc.subcore_barrier()

    crow = (cid * NS + sid) * NCH

    pltpu.sync_copy(edges_ref.at[crow + 0], e0)
    pltpu.sync_copy(edges_ref.at[crow + 1], e1)
    pltpu.async_copy(xw_ref.at[e0.at[0]], r0, g0)
    pltpu.async_copy(xw_ref.at[e1.at[0]], r1, g1)
    pltpu.async_copy(edges_ref.at[crow + 2], e2, es2)
    pltpu.async_copy(edges_ref.at[crow + 3], e3, es3)

    def _ring(g, _):
        for b in range(4):
            j = g * 4 + b
            p = b % 2
            row = rows[p]
            eb = ebuf[b]
            pltpu.make_async_copy(xw_ref.at[eb.at[0]], row, gsem[p]).wait()
            def _scale(jj, _2, row=row, eb=eb):
                sl16 = pl.ds(pl.multiple_of(jj * L, L), L)
                wv = plsc.bitcast(eb[2, sl16], jnp.float32)
                for rr in range(L):
                    w1 = wv[rr]
                    r = jj * L + rr
                    for f in range(FPV):
                        sl = pl.ds(f * L, L)
                        row[r, sl] = row[r, sl] * w1
                return 0
            lax.fori_loop(0, CH // L, _scale, 0)
            pltpu.async_copy(row, agg_sp.at[eb.at[1]], ssem[p], add=True)

            @pl.when(j < NCH - 2)
            def _(j=j, p=p, b=b, row=row):
                eb2 = ebuf[(b + 2) % 4]
                pltpu.make_async_copy(edges_ref.at[crow], eb2,
                                      esem[(b + 2) % 4]).wait()
                pltpu.make_async_copy(row, agg_sp.at[eb2.at[1]],
                                      ssem[p]).wait()
                pltpu.async_copy(xw_ref.at[eb2.at[0]], row, gsem[p])

                @pl.when(j < NCH - 4)
                def _():
                    pltpu.async_copy(edges_ref.at[crow + j + 4], ebuf[b],
                                     esem[b])
        return 0

    lax.fori_loop(0, NCH // 4, _ring, 0)
    for p in range(2):
        pltpu.make_async_copy(rows[p], agg_sp.at[e0.at[1]], ssem[p]).wait()
    plsc.subcore_barrier()

    pltpu.sync_copy(agg_sp.at[pl.ds(sid * RPT, RPT)],
                    out_ref.at[cid, pl.ds(sid * RPT, RPT)])


_agg_kernel = pl.kernel(
    _agg_body,
    out_type=jax.ShapeDtypeStruct((NC, N_PAD, D), jnp.float32),
    mesh=_mesh,
    compiler_params=_sc_params,
    scratch_types=[
        pltpu.VMEM_SHARED((N_PAD, D), jnp.float32),
        pltpu.VMEM((CH, D), jnp.float32),
        pltpu.VMEM((CH, D), jnp.float32),
        pltpu.VMEM((3, CH), jnp.int32),
        pltpu.VMEM((3, CH), jnp.int32),
        pltpu.VMEM((3, CH), jnp.int32),
        pltpu.VMEM((3, CH), jnp.int32),
        pltpu.SemaphoreType.DMA,
        pltpu.SemaphoreType.DMA,
        pltpu.SemaphoreType.DMA,
        pltpu.SemaphoreType.DMA,
        pltpu.SemaphoreType.DMA,
        pltpu.SemaphoreType.DMA,
        pltpu.SemaphoreType.DMA,
        pltpu.SemaphoreType.DMA,
    ],
)


_R = 400


def _mm_body(x_ref, w_ref, dinv_ref, o_ref):
    o_ref[...] = jnp.dot(x_ref[...], w_ref[...],
                         preferred_element_type=jnp.float32) * dinv_ref[...]


def _tc_matmul(x, W, dinv):
    return pl.pallas_call(
        _mm_body,
        grid=(N // _R,),
        in_specs=[pl.BlockSpec((_R, D), lambda i: (i, 0)),
                  pl.BlockSpec((D, D), lambda i: (0, 0)),
                  pl.BlockSpec((_R, 1), lambda i: (i, 0))],
        out_specs=pl.BlockSpec((_R, D), lambda i: (i, 0)),
        out_shape=jax.ShapeDtypeStruct((N, D), jnp.float32),
    )(x, W, dinv)


def _combine_mm_body(agg_ref, xw_ref, dinv_ref, b_ref, w_ref, o_ref):
    h = (agg_ref[0] + agg_ref[1] + xw_ref[...]) * dinv_ref[...] + b_ref[...]
    h = jnp.maximum(h, 0.0)
    o_ref[...] = jnp.dot(h, w_ref[...],
                         preferred_element_type=jnp.float32) * dinv_ref[...]


def _combine_mm(agg, xw, dinv, b, W):
    return pl.pallas_call(
        _combine_mm_body,
        grid=(N // _R,),
        in_specs=[pl.BlockSpec((NC, _R, D), lambda i: (0, i, 0)),
                  pl.BlockSpec((_R, D), lambda i: (i, 0)),
                  pl.BlockSpec((_R, 1), lambda i: (i, 0)),
                  pl.BlockSpec((1, D), lambda i: (0, 0)),
                  pl.BlockSpec((D, D), lambda i: (0, 0))],
        out_specs=pl.BlockSpec((_R, D), lambda i: (i, 0)),
        out_shape=jax.ShapeDtypeStruct((N, D), jnp.float32),
    )(agg, xw, dinv, b, W)


def _final_body(agg_ref, xw_ref, dinv_ref, b_ref, o_ref):
    o_ref[...] = ((agg_ref[0] + agg_ref[1] + xw_ref[...]) * dinv_ref[...]
                  + b_ref[...])


def _final(agg, xw, dinv, b):
    return pl.pallas_call(
        _final_body,
        grid=(N // _R,),
        in_specs=[pl.BlockSpec((NC, _R, D), lambda i: (0, i, 0)),
                  pl.BlockSpec((_R, D), lambda i: (i, 0)),
                  pl.BlockSpec((_R, 1), lambda i: (i, 0)),
                  pl.BlockSpec((1, D), lambda i: (0, 0))],
        out_specs=pl.BlockSpec((_R, D), lambda i: (i, 0)),
        out_shape=jax.ShapeDtypeStruct((N, D), jnp.float32),
    )(agg, xw, dinv, b)


def kernel(x, edge_index, edge_weights, W1, b1, W2, b2):
    pad = E_PAD - E
    src_p = jnp.concatenate([edge_index[0], jnp.zeros((pad,), jnp.int32)])
    dst_p = jnp.concatenate([edge_index[1], jnp.zeros((pad,), jnp.int32)])
    ew_p = jnp.concatenate([edge_weights, jnp.zeros((pad,), jnp.float32)])
    ew_bits = jax.lax.bitcast_convert_type(ew_p, jnp.int32)
    edges = jnp.stack([src_p.reshape(NCHT, CH), dst_p.reshape(NCHT, CH),
                       ew_bits.reshape(NCHT, CH)], axis=1)

    dinv = _dinv_kernel(dst_p, ew_p)
    dinv2 = dinv[:N, None]
    b1r = b1[None, :]
    b2r = b2[None, :]

    xws1 = _tc_matmul(x, W1, dinv2)
    agg1 = _agg_kernel(xws1, edges)
    xws2 = _combine_mm(agg1, xws1, dinv2, b1r, W2)
    agg2 = _agg_kernel(xws2, edges)
    return _final(agg2, xws2, dinv2, b2r)
